# Initial kernel scaffold; baseline (speedup 1.0000x reference)
import jax
import jax.numpy as jnp
from jax.experimental import pallas as pl


def kernel(inputs, table, W, b):
    raise NotImplementedError("write your pallas kernel here")


# speedup vs baseline: 5.1027x; 5.1027x over previous
import functools

import jax
import jax.numpy as jnp
from jax import lax
from jax.experimental import pallas as pl
from jax.experimental.pallas import tpu as pltpu
from jax.experimental.pallas import tpu_sc as plsc

_B = 16384
_L = 200
_N = _B * _L
_D = 4
_NC = 2
_NS = 16
_NW = _NC * _NS
_PER_W = _N // _NW
_C = 10240
_NCHUNK = _PER_W // _C
_G = _C // 4


def _sc_lookup(idx, pk):
  mesh = plsc.VectorSubcoreMesh(core_axis_name="c", subcore_axis_name="s")

  @functools.partial(
      pl.kernel,
      mesh=mesh,
      compiler_params=pltpu.CompilerParams(needs_layout_passes=False),
      out_type=jax.ShapeDtypeStruct((_N * _D,), jnp.float32),
      scratch_types=[
          pltpu.VMEM((32,), jnp.float32),
          pltpu.VMEM((_C,), jnp.int32),
          pltpu.VMEM((_C * _D,), jnp.float32),
      ],
  )
  def k(idx_hbm, pk_hbm, out_hbm, pk_v, idx_v, out_v):
    wid = lax.axis_index("s") * _NC + lax.axis_index("c")
    pltpu.sync_copy(pk_hbm, pk_v)
    lane = lax.iota(jnp.int32, 16)
    cmod = lane & 3
    quad = lane >> 2
    w0 = plsc.load_gather(pk_v, [8 + cmod])
    w1 = plsc.load_gather(pk_v, [12 + cmod])
    bc = plsc.load_gather(pk_v, [16 + cmod])
    luts = []
    for v in range(3):
      t0 = plsc.load_gather(pk_v, [jnp.full((16,), 2 + 2 * v, jnp.int32)])
      t1 = plsc.load_gather(pk_v, [jnp.full((16,), 3 + 2 * v, jnp.int32)])
      luts.append(t0 * w0 + t1 * w1 + bc)
    base = wid * _PER_W
    for c in range(_NCHUNK):
      off = base + c * _C
      pltpu.sync_copy(idx_hbm.at[pl.ds(off, _C)], idx_v)

      def body(j, carry):
        idxr = plsc.load_gather(idx_v, [j * 4 + quad])
        o = jnp.where(idxr == 0, luts[0],
                      jnp.where(idxr == 1, luts[1], luts[2]))
        out_v[pl.ds(j * 16, 16)] = o
        return carry

      lax.fori_loop(0, _G, body, 0)
      pltpu.sync_copy(out_v, out_hbm.at[pl.ds(off * _D, _C * _D)])

  return k(idx, pk)


def kernel(inputs, table, W, b):
  idx = inputs.reshape(-1).astype(jnp.int32)
  pk = jnp.zeros((32,), jnp.float32)
  pk = pk.at[2:8].set(table.reshape(-1).astype(jnp.float32))
  pk = pk.at[8:16].set(W.reshape(-1).astype(jnp.float32))
  pk = pk.at[16:20].set(b.astype(jnp.float32))
  out = _sc_lookup(idx, pk)
  return out.reshape(_B, _L, _D)


# speedup vs baseline: 78.3332x; 15.3512x over previous
import functools

import jax
import jax.numpy as jnp
from jax import lax
from jax.experimental import pallas as pl
from jax.experimental.pallas import tpu as pltpu
from jax.experimental.pallas import tpu_sc as plsc

_B = 16384
_L = 200
_D = 4
_NC = 2
_NS = 16
_NW = _NC * _NS
_BG = 4
_LG = _NW // _BG
_BC = _B // _BG
_LPW = _L // _LG
_GRP = _BC // 16


def _sc_lookup(idx2, pk):
  mesh = plsc.VectorSubcoreMesh(core_axis_name="c", subcore_axis_name="s")

  @functools.partial(
      pl.kernel,
      mesh=mesh,
      compiler_params=pltpu.CompilerParams(needs_layout_passes=False),
      out_type=jax.ShapeDtypeStruct((_L * _D, _B), jnp.float32),
      scratch_types=[
          pltpu.VMEM((32,), jnp.float32),
          pltpu.VMEM((1, _BC), jnp.int32),
          pltpu.VMEM((_D, _BC), jnp.float32),
      ],
  )
  def k(idx_hbm, pk_hbm, out_hbm, pk_v, idx_v, out_v):
    wid = lax.axis_index("s") * _NC + lax.axis_index("c")
    lgrp = wid // _BG
    b0 = (wid % _BG) * _BC
    pltpu.sync_copy(pk_hbm, pk_v)
    def splat(i):
      return plsc.load_gather(pk_v, [jnp.full((16,), i, jnp.int32)])

    lut = [[splat(2 + 2 * v) * splat(8 + c) + splat(3 + 2 * v) * splat(12 + c)
            + splat(16 + c) for c in range(_D)] for v in range(3)]

    for kk in range(_LPW):
      l = lgrp * _LPW + kk
      pltpu.sync_copy(idx_hbm.at[pl.ds(l, 1), pl.ds(b0, _BC)], idx_v)

      def body(g, carry):
        iv = idx_v[0, pl.ds(g * 16, 16)]
        m0 = iv == 0
        m1 = iv == 1
        for c in range(_D):
          out_v[c, pl.ds(g * 16, 16)] = jnp.where(
              m0, lut[0][c], jnp.where(m1, lut[1][c], lut[2][c]))
        return carry

      lax.fori_loop(0, _GRP, body, 0)
      pltpu.sync_copy(out_v, out_hbm.at[pl.ds(l * _D, _D), pl.ds(b0, _BC)])

  return k(idx2, pk)


def kernel(inputs, table, W, b):
  idx2 = inputs.T.astype(jnp.int32)
  pk = jnp.zeros((32,), jnp.float32)
  pk = pk.at[2:8].set(table.reshape(-1).astype(jnp.float32))
  pk = pk.at[8:16].set(W.reshape(-1).astype(jnp.float32))
  pk = pk.at[16:20].set(b.astype(jnp.float32))
  out = _sc_lookup(idx2, pk)
  return out.reshape(_L, _D, _B).transpose(2, 0, 1)


# speedup vs baseline: 111.6665x; 1.4255x over previous
import functools

import jax
import jax.numpy as jnp
from jax import lax
from jax.experimental import pallas as pl
from jax.experimental.pallas import tpu as pltpu
from jax.experimental.pallas import tpu_sc as plsc

_B = 16384
_L = 200
_D = 4
_NC = 2
_NS = 16
_NW = _NC * _NS
_BG = 4
_LG = _NW // _BG
_BC = _B // _BG
_LPW = _L // _LG
_BT = _BC // 128


def _sc_lookup(idx2, pk):
  mesh = plsc.VectorSubcoreMesh(core_axis_name="c", subcore_axis_name="s")

  @functools.partial(
      pl.kernel,
      mesh=mesh,
      compiler_params=pltpu.CompilerParams(needs_layout_passes=False),
      out_type=jax.ShapeDtypeStruct((_L * _D * (_B // 128), 128), jnp.float32),
      scratch_types=[
          pltpu.VMEM((32,), jnp.float32),
          pltpu.VMEM((1, _BC), jnp.int32),
          pltpu.VMEM((_BT * _D, 128), jnp.float32),
      ],
  )
  def k(idx_hbm, pk_hbm, out_hbm, pk_v, idx_v, out_v):
    wid = lax.axis_index("s") * _NC + lax.axis_index("c")
    lgrp = wid // _BG
    bgrp = wid % _BG
    b0 = bgrp * _BC
    pltpu.sync_copy(pk_hbm, pk_v)
    def splat(i):
      return plsc.load_gather(pk_v, [jnp.full((16,), i, jnp.int32)])

    lut = [[splat(2 + 2 * v) * splat(8 + c) + splat(3 + 2 * v) * splat(12 + c)
            + splat(16 + c) for c in range(_D)] for v in range(3)]

    for kk in range(_LPW):
      l = lgrp * _LPW + kk
      pltpu.sync_copy(idx_hbm.at[pl.ds(l, 1), pl.ds(b0, _BC)], idx_v)

      def body(bt, carry):
        for m in range(8):
          iv = idx_v[0, pl.ds(bt * 128 + m * 16, 16)]
          m0 = iv == 0
          m1 = iv == 1
          for c in range(_D):
            out_v[bt * _D + c, pl.ds(m * 16, 16)] = jnp.where(
                m0, lut[0][c], jnp.where(m1, lut[1][c], lut[2][c]))
        return carry

      lax.fori_loop(0, _BT, body, 0)
      pltpu.sync_copy(
          out_v, out_hbm.at[pl.ds(l * _D * (_B // 128) + bgrp * _BT * _D,
                                  _BT * _D), :])

  return k(idx2, pk)


def kernel(inputs, table, W, b):
  idx2 = inputs.T.astype(jnp.int32)
  pk = jnp.zeros((32,), jnp.float32)
  pk = pk.at[2:8].set(table.reshape(-1).astype(jnp.float32))
  pk = pk.at[8:16].set(W.reshape(-1).astype(jnp.float32))
  pk = pk.at[16:20].set(b.astype(jnp.float32))
  out = _sc_lookup(idx2, pk)
  out = out.reshape(_L, _B // 128, _D, 128)
  return out.transpose(1, 3, 0, 2).reshape(_B, _L, _D)


# speedup vs baseline: 167.6570x; 1.5014x over previous
import functools

import jax
import jax.numpy as jnp
from jax import lax
from jax.experimental import pallas as pl
from jax.experimental.pallas import tpu as pltpu
from jax.experimental.pallas import tpu_sc as plsc

_B = 16384
_L = 200
_D = 4
_NC = 2
_NS = 16
_NW = _NC * _NS
_BG = 4
_LG = _NW // _BG
_BC = _B // _BG
_LPW = _L // _LG
_BT = _BC // 128


def _sc_lookup(idx2, pk):
  mesh = plsc.VectorSubcoreMesh(core_axis_name="c", subcore_axis_name="s")

  @functools.partial(
      pl.kernel,
      mesh=mesh,
      compiler_params=pltpu.CompilerParams(needs_layout_passes=False),
      out_type=jax.ShapeDtypeStruct((_L * _D * (_B // 128), 128), jnp.float32),
      scratch_types=[
          pltpu.VMEM((32,), jnp.float32),
          pltpu.VMEM((1, _BC), jnp.int32),
          pltpu.VMEM((1, _BC), jnp.int32),
          pltpu.VMEM((_BT * _D, 128), jnp.float32),
          pltpu.VMEM((_BT * _D, 128), jnp.float32),
          pltpu.SemaphoreType.DMA,
          pltpu.SemaphoreType.DMA,
          pltpu.SemaphoreType.DMA,
          pltpu.SemaphoreType.DMA,
      ],
  )
  def k(idx_hbm, pk_hbm, out_hbm, pk_v, idx_v0, idx_v1, out_v0, out_v1,
        si0, si1, so0, so1):
    wid = lax.axis_index("s") * _NC + lax.axis_index("c")
    lgrp = wid // _BG
    bgrp = wid % _BG
    b0 = bgrp * _BC
    pltpu.sync_copy(pk_hbm, pk_v)
    def splat(i):
      return plsc.load_gather(pk_v, [jnp.full((16,), i, jnp.int32)])

    lut = [[splat(2 + 2 * v) * splat(8 + c) + splat(3 + 2 * v) * splat(12 + c)
            + splat(16 + c) for c in range(_D)] for v in range(3)]

    idx_bufs, out_bufs = (idx_v0, idx_v1), (out_v0, out_v1)
    in_sems, out_sems = (si0, si1), (so0, so1)

    def idx_dma(kk, p):
      l = lgrp * _LPW + kk
      return pltpu.async_copy(
          idx_hbm.at[pl.ds(l, 1), pl.ds(b0, _BC)], idx_bufs[p], in_sems[p])

    def out_dma(kk, p):
      l = lgrp * _LPW + kk
      return pltpu.async_copy(
          out_bufs[p],
          out_hbm.at[pl.ds(l * _D * (_B // 128) + bgrp * _BT * _D,
                           _BT * _D), :],
          out_sems[p])

    in_h = [idx_dma(0, 0), None]
    out_h = [None, None]
    for kk in range(_LPW):
      p = kk & 1
      if kk + 1 < _LPW:
        in_h[1 - p] = idx_dma(kk + 1, 1 - p)
      in_h[p].wait()
      if out_h[p] is not None:
        out_h[p].wait()
      idx_v, out_v = idx_bufs[p], out_bufs[p]

      def body(bt, carry):
        for m in range(8):
          iv = idx_v[0, pl.ds(bt * 128 + m * 16, 16)]
          m0 = iv == 0
          m1 = iv == 1
          for c in range(_D):
            out_v[bt * _D + c, pl.ds(m * 16, 16)] = jnp.where(
                m0, lut[0][c], jnp.where(m1, lut[1][c], lut[2][c]))
        return carry

      lax.fori_loop(0, _BT, body, 0)
      out_h[p] = out_dma(kk, p)
    for p in ((_LPW - 1) & 1, _LPW & 1):
      if out_h[p] is not None:
        out_h[p].wait()

  return k(idx2, pk)


def kernel(inputs, table, W, b):
  idx2 = inputs.T.astype(jnp.int32)
  pk = jnp.zeros((32,), jnp.float32)
  pk = pk.at[2:8].set(table.reshape(-1).astype(jnp.float32))
  pk = pk.at[8:16].set(W.reshape(-1).astype(jnp.float32))
  pk = pk.at[16:20].set(b.astype(jnp.float32))
  out = _sc_lookup(idx2, pk)
  out = out.reshape(_L, _B // 128, _D, 128)
  return out.transpose(1, 3, 0, 2).reshape(_B, _L, _D)
